# baseline (speedup 1.0000x reference)
import functools

import jax
import jax.numpy as jnp
from jax import lax
from jax.experimental import pallas as pl
from jax.experimental.pallas import tpu as pltpu
from jax.experimental.pallas import tpu_sc as plsc

N = 16384
D = 64
K = 8192
MQ = 256
KC = 4096
NQT = N // MQ


def _argmin_body(f_ref, e_ref, ids_ref, lsum_ref):
    i = pl.program_id(0)
    f = f_ref[0]
    fsq = jnp.sum(f * f, axis=0, keepdims=True)
    fb = (f * 2.0).astype(jnp.bfloat16)
    best_sel = jnp.full((MQ,), jnp.inf, dtype=jnp.float32)
    best_round = jnp.full((MQ,), jnp.inf, dtype=jnp.float32)
    best_idx = jnp.zeros((MQ,), dtype=jnp.int32)
    for kc in range(K // KC):
        e = e_ref[kc * KC:(kc + 1) * KC, :]
        esq = jnp.sum(e * e, axis=1, keepdims=True)
        eb = e.astype(jnp.bfloat16)
        conv = lax.dot_general(eb, fb, (((1,), (0,)), ((), ())),
                               preferred_element_type=jnp.float32)
        d2 = (fsq - conv) + esq
        lv = jnp.min(d2, axis=0)
        li = jnp.argmin(d2, axis=0).astype(jnp.int32)
        upd = lv < best_round
        best_idx = jnp.where(upd, li + kc * KC, best_idx)
        best_sel = jnp.where(upd, lv, best_sel)
        best_round = jnp.where(upd, lv, best_round)
        best_round = best_round.astype(jnp.bfloat16).astype(jnp.float32)
    ids_ref[0, 0, :] = best_idx

    @pl.when(i == 0)
    def _():
        lsum_ref[...] = jnp.zeros((1, 1), jnp.float32)

    lsum_ref[...] += jnp.sum(best_sel).reshape(1, 1)


def _nearest_codes(x3, embed):
    ntile = x3.shape[2] // MQ
    ids3d, lsum = pl.pallas_call(
        _argmin_body,
        grid=(NQT,),
        in_specs=[
            pl.BlockSpec((1, D, MQ), lambda i: (i // ntile, 0, i % ntile)),
            pl.BlockSpec((K, D), lambda i: (0, 0)),
        ],
        out_specs=[
            pl.BlockSpec((1, 1, MQ), lambda i: (i, 0, 0)),
            pl.BlockSpec((1, 1), lambda i: (0, 0)),
        ],
        out_shape=[
            jax.ShapeDtypeStruct((NQT, 1, MQ), jnp.int32),
            jax.ShapeDtypeStruct((1, 1), jnp.float32),
        ],
    )(x3, embed)
    return ids3d.reshape(N), lsum[0, 0] / (N * D)


@functools.cache
def _make_gather():
    info = plsc.get_sparse_core_info()
    nw = info.num_cores * info.num_subcores
    b_per_w = N // nw
    mesh = plsc.VectorSubcoreMesh(core_axis_name="c", subcore_axis_name="s")

    @functools.partial(
        pl.kernel,
        mesh=mesh,
        compiler_params=pltpu.CompilerParams(use_tc_tiling_on_sc=False),
        out_type=jax.ShapeDtypeStruct((N, D), jnp.float32),
        scratch_types=[
            pltpu.VMEM((b_per_w,), jnp.int32),
            pltpu.VMEM((b_per_w, D), jnp.float32),
            pltpu.SemaphoreType.DMA,
        ],
    )
    def gather(table_hbm, idx_hbm, out_hbm, idx_v, rows_v, sem):
        wid = lax.axis_index("s") * info.num_cores + lax.axis_index("c")
        base = wid * b_per_w
        pltpu.sync_copy(idx_hbm.at[pl.ds(base, b_per_w)], idx_v)
        pltpu.async_copy(table_hbm.at[idx_v], rows_v, sem).wait()
        pltpu.sync_copy(rows_v, out_hbm.at[pl.ds(base, b_per_w)])

    return gather


def kernel(input, embed):
    b, c, h, w = input.shape
    x3 = input.reshape(b, c, h * w)
    ids, loss = _nearest_codes(x3, embed)
    vqs = _make_gather()(embed, ids)
    ste_out = vqs.reshape(b, c, h, w)
    ids3 = ids.reshape(b, h, w)
    return ste_out, loss, ids3


# speedup vs baseline: 1.1952x; 1.1952x over previous
import functools

import jax
import jax.numpy as jnp
from jax import lax
from jax.experimental import pallas as pl
from jax.experimental.pallas import tpu as pltpu
from jax.experimental.pallas import tpu_sc as plsc

N = 16384
D = 64
K = 8192
MQ = 512
KC = 4096
NQT = N // MQ


def _argmin_body(f_ref, e_ref, ids_ref, lsum_ref):
    i = pl.program_id(0)
    f = f_ref[0]
    fsq = jnp.sum(f * f, axis=0, keepdims=True)
    fb = (f * 2.0).astype(jnp.bfloat16)
    best_sel = jnp.full((MQ,), jnp.inf, dtype=jnp.float32)
    best_round = jnp.full((MQ,), jnp.inf, dtype=jnp.float32)
    best_idx = jnp.zeros((MQ,), dtype=jnp.int32)
    for kc in range(K // KC):
        e = e_ref[kc * KC:(kc + 1) * KC, :]
        esq = jnp.sum(e * e, axis=1, keepdims=True)
        eb = e.astype(jnp.bfloat16)
        conv = lax.dot_general(eb, fb, (((1,), (0,)), ((), ())),
                               preferred_element_type=jnp.float32)
        d2 = (fsq - conv) + esq
        lv = jnp.min(d2, axis=0)
        li = jnp.argmin(d2, axis=0).astype(jnp.int32)
        upd = lv < best_round
        best_idx = jnp.where(upd, li + kc * KC, best_idx)
        best_sel = jnp.where(upd, lv, best_sel)
        best_round = jnp.where(upd, lv, best_round)
        best_round = best_round.astype(jnp.bfloat16).astype(jnp.float32)
    ids_ref[0, 0, :] = best_idx

    @pl.when(i == 0)
    def _():
        lsum_ref[...] = jnp.zeros((1, 1), jnp.float32)

    lsum_ref[...] += jnp.sum(best_sel).reshape(1, 1)


def _nearest_codes(x3, embed):
    ntile = x3.shape[2] // MQ
    ids3d, lsum = pl.pallas_call(
        _argmin_body,
        grid=(NQT,),
        in_specs=[
            pl.BlockSpec((1, D, MQ), lambda i: (i // ntile, 0, i % ntile)),
            pl.BlockSpec((K, D), lambda i: (0, 0)),
        ],
        out_specs=[
            pl.BlockSpec((1, 1, MQ), lambda i: (i, 0, 0)),
            pl.BlockSpec((1, 1), lambda i: (0, 0)),
        ],
        out_shape=[
            jax.ShapeDtypeStruct((NQT, 1, MQ), jnp.int32),
            jax.ShapeDtypeStruct((1, 1), jnp.float32),
        ],
    )(x3, embed)
    return ids3d.reshape(N), lsum[0, 0] / (N * D)


@functools.cache
def _make_gather():
    info = plsc.get_sparse_core_info()
    nw = info.num_cores * info.num_subcores
    b_per_w = N // nw
    mesh = plsc.VectorSubcoreMesh(core_axis_name="c", subcore_axis_name="s")

    @functools.partial(
        pl.kernel,
        mesh=mesh,
        compiler_params=pltpu.CompilerParams(use_tc_tiling_on_sc=False),
        out_type=jax.ShapeDtypeStruct((N, D), jnp.float32),
        scratch_types=[
            pltpu.VMEM((b_per_w,), jnp.int32),
            pltpu.VMEM((b_per_w, D), jnp.float32),
            pltpu.SemaphoreType.DMA,
        ],
    )
    def gather(table_hbm, idx_hbm, out_hbm, idx_v, rows_v, sem):
        wid = lax.axis_index("s") * info.num_cores + lax.axis_index("c")
        base = wid * b_per_w
        pltpu.sync_copy(idx_hbm.at[pl.ds(base, b_per_w)], idx_v)
        pltpu.async_copy(table_hbm.at[idx_v], rows_v, sem).wait()
        pltpu.sync_copy(rows_v, out_hbm.at[pl.ds(base, b_per_w)])

    return gather


def kernel(input, embed):
    b, c, h, w = input.shape
    x3 = input.reshape(b, c, h * w)
    ids, loss = _nearest_codes(x3, embed)
    vqs = _make_gather()(embed, ids)
    vqs4 = jnp.transpose(vqs.reshape(b, h, w, c), (0, 3, 1, 2))
    ste_out = input + (vqs4 - input)
    ids3 = ids.reshape(b, h, w)
    return ste_out, loss, ids3


# speedup vs baseline: 1.2878x; 1.0775x over previous
import functools

import jax
import jax.numpy as jnp
from jax import lax
from jax.experimental import pallas as pl
from jax.experimental.pallas import tpu as pltpu
from jax.experimental.pallas import tpu_sc as plsc

N = 16384
D = 64
K = 8192
MQ = 1024
KC = 4096
NQT = N // MQ


def _argmin_body(f_ref, e_ref, ids_ref, lsum_ref):
    i = pl.program_id(0)
    f = f_ref[0]
    fsq = jnp.sum(f * f, axis=0, keepdims=True)
    fb = (f * 2.0).astype(jnp.bfloat16)
    best_sel = jnp.full((MQ,), jnp.inf, dtype=jnp.float32)
    best_round = jnp.full((MQ,), jnp.inf, dtype=jnp.float32)
    best_idx = jnp.zeros((MQ,), dtype=jnp.int32)
    for kc in range(K // KC):
        e = e_ref[kc * KC:(kc + 1) * KC, :]
        esq = jnp.sum(e * e, axis=1, keepdims=True)
        eb = e.astype(jnp.bfloat16)
        conv = lax.dot_general(eb, fb, (((1,), (0,)), ((), ())),
                               preferred_element_type=jnp.float32)
        d2 = (fsq - conv) + esq
        lv = jnp.min(d2, axis=0)
        li = jnp.argmin(d2, axis=0).astype(jnp.int32)
        upd = lv < best_round
        best_idx = jnp.where(upd, li + kc * KC, best_idx)
        best_sel = jnp.where(upd, lv, best_sel)
        best_round = jnp.where(upd, lv, best_round)
        best_round = best_round.astype(jnp.bfloat16).astype(jnp.float32)
    ids_ref[0, 0, :] = best_idx

    @pl.when(i == 0)
    def _():
        lsum_ref[...] = jnp.zeros((1, 1), jnp.float32)

    lsum_ref[...] += jnp.sum(best_sel).reshape(1, 1)


def _nearest_codes(x3, embed):
    ntile = x3.shape[2] // MQ
    ids3d, lsum = pl.pallas_call(
        _argmin_body,
        grid=(NQT,),
        in_specs=[
            pl.BlockSpec((1, D, MQ), lambda i: (i // ntile, 0, i % ntile)),
            pl.BlockSpec((K, D), lambda i: (0, 0)),
        ],
        out_specs=[
            pl.BlockSpec((1, 1, MQ), lambda i: (i, 0, 0)),
            pl.BlockSpec((1, 1), lambda i: (0, 0)),
        ],
        out_shape=[
            jax.ShapeDtypeStruct((NQT, 1, MQ), jnp.int32),
            jax.ShapeDtypeStruct((1, 1), jnp.float32),
        ],
    )(x3, embed)
    return ids3d.reshape(N), lsum[0, 0] / (N * D)


@functools.cache
def _make_gather():
    info = plsc.get_sparse_core_info()
    nw = info.num_cores * info.num_subcores
    b_per_w = N // nw
    mesh = plsc.VectorSubcoreMesh(core_axis_name="c", subcore_axis_name="s")

    @functools.partial(
        pl.kernel,
        mesh=mesh,
        compiler_params=pltpu.CompilerParams(use_tc_tiling_on_sc=False),
        out_type=jax.ShapeDtypeStruct((N, D), jnp.float32),
        scratch_types=[
            pltpu.VMEM((b_per_w,), jnp.int32),
            pltpu.VMEM((b_per_w, D), jnp.float32),
            pltpu.SemaphoreType.DMA,
        ],
    )
    def gather(table_hbm, idx_hbm, out_hbm, idx_v, rows_v, sem):
        wid = lax.axis_index("s") * info.num_cores + lax.axis_index("c")
        base = wid * b_per_w
        pltpu.sync_copy(idx_hbm.at[pl.ds(base, b_per_w)], idx_v)
        pltpu.async_copy(table_hbm.at[idx_v], rows_v, sem).wait()
        pltpu.sync_copy(rows_v, out_hbm.at[pl.ds(base, b_per_w)])

    return gather


def kernel(input, embed):
    b, c, h, w = input.shape
    x3 = input.reshape(b, c, h * w)
    ids, loss = _nearest_codes(x3, embed)
    vqs = _make_gather()(embed, ids)
    vqs4 = jnp.transpose(vqs.reshape(b, h, w, c), (0, 3, 1, 2))
    ste_out = input + (vqs4 - input)
    ids3 = ids.reshape(b, h, w)
    return ste_out, loss, ids3
